# baseline (speedup 1.0000x reference)
import functools

import jax
import jax.numpy as jnp
from jax import lax
from jax.experimental import pallas as pl
from jax.experimental.pallas import tpu as pltpu
from jax.experimental.pallas import tpu_sc as plsc

B = 16384
NS = 26
ND = 13
VOCAB = 100000
EMB = 16
EMB_FLAT = NS * EMB
NROWS = B * NS
NW = 32
ROWS_PER_W = NROWS // NW
SUB = 128
NSUB = ROWS_PER_W // SUB
GROUP = 13
CHUNK = SUB * GROUP
NCHUNK = ROWS_PER_W // CHUNK

@functools.cache
def _make_gather_sc():
    mesh = plsc.VectorSubcoreMesh(core_axis_name="c", subcore_axis_name="s")

    @functools.partial(
        pl.kernel,
        mesh=mesh,
        out_type=jax.ShapeDtypeStruct((NROWS, EMB), jnp.float32),
        compiler_params=pltpu.CompilerParams(use_tc_tiling_on_sc=False),
        scratch_types=[
            pltpu.VMEM((NSUB, SUB), jnp.int32),
            pltpu.VMEM((CHUNK, EMB), jnp.float32),
            pltpu.VMEM((CHUNK, EMB), jnp.float32),
            pltpu.SemaphoreType.DMA,
            pltpu.SemaphoreType.DMA,
            pltpu.SemaphoreType.DMA,
            pltpu.SemaphoreType.DMA,
        ],
    )
    def _gather_sc(table_hbm, idx_hbm, out_hbm, idx_v, buf0, buf1, g0, g1, o0, o1):
        wid = lax.axis_index("s") * 2 + lax.axis_index("c")
        base = wid * ROWS_PER_W
        pltpu.sync_copy(idx_hbm.at[wid], idx_v)
        bufs = (buf0, buf1)
        gsem = (g0, g1)
        osem = (o0, o1)
        def fire(grp, b):
            return [
                pltpu.async_copy(
                    table_hbm.at[idx_v.at[grp * GROUP + i]],
                    bufs[b].at[pl.ds(i * SUB, SUB)], gsem[b])
                for i in range(GROUP)
            ]

        gath = [None, None]
        outc = [None, None]
        gath[0] = fire(0, 0)
        for j in range(NCHUNK):
            cur = j & 1
            nxt = 1 - cur
            if j + 1 < NCHUNK:
                if outc[nxt] is not None:
                    outc[nxt].wait()
                gath[nxt] = fire(j + 1, nxt)
            for c in gath[cur]:
                c.wait()
            outc[cur] = pltpu.async_copy(
                bufs[cur], out_hbm.at[pl.ds(base + j * CHUNK, CHUNK)], osem[cur])
        outc[0].wait()
        outc[1].wait()

    return _gather_sc


VSUB = 10000
FLAT_ROWS = NS * VOCAB * EMB // 128
ROWS_PER_F = VOCAB * EMB // 128


def _retile_tc_body(x_ref, o_ref):
    eye = jnp.eye(EMB, dtype=jnp.float32)
    for f01 in range(2):
        for m in range(8):
            xm = x_ref[f01, :, m * ROWS_PER_F:(m + 1) * ROWS_PER_F]
            y = jax.lax.dot_general(
                xm, eye, (((0,), (0,)), ((), ())),
                preferred_element_type=jnp.float32)
            o_ref[f01 * ROWS_PER_F:(f01 + 1) * ROWS_PER_F,
                  EMB * m:EMB * (m + 1)] = y


_retile_tc = pl.pallas_call(
    _retile_tc_body,
    grid=(NS // 2,),
    in_specs=[pl.BlockSpec((2, EMB, VOCAB), lambda k: (k, 0, 0))],
    out_specs=pl.BlockSpec((2 * ROWS_PER_F, 128), lambda k: (k, 0)),
    out_shape=jax.ShapeDtypeStruct((FLAT_ROWS, 128), jnp.float32),
    compiler_params=pltpu.CompilerParams(vmem_limit_bytes=100 * 1024 * 1024),
)


BB = 2048


def _deepfm_tc_body(embs_ref, dx_ref, w1e_ref, w1d_ref, b1_ref, w2_ref, b2_ref,
                    wfe_ref, wfd_ref, bf_ref, wd_ref, bd_ref, wo_ref, bo_ref,
                    out_ref):
    e = embs_ref[...]
    dx = dx_ref[...]
    x = jnp.dot(e, w1e_ref[...], preferred_element_type=jnp.float32)
    x = x + jnp.dot(dx, w1d_ref[...], preferred_element_type=jnp.float32)
    x = jnp.maximum(x + b1_ref[...], 0.0)
    x = jnp.dot(x, w2_ref[...], preferred_element_type=jnp.float32)
    x = jnp.maximum(x + b2_ref[...], 0.0)
    fm = jnp.dot(e, wfe_ref[...], preferred_element_type=jnp.float32)
    fm = fm + jnp.dot(dx, wfd_ref[...], preferred_element_type=jnp.float32)
    fm = fm + bf_ref[...]
    deep = jnp.dot(x, wd_ref[...], preferred_element_type=jnp.float32)
    deep = deep + bd_ref[...]
    wo = wo_ref[...]
    z = fm * wo[0:1, :] + deep * wo[1:2, :] + bo_ref[...]
    out_ref[...] = jax.nn.sigmoid(z)


def _full(shape):
    return pl.BlockSpec(shape, lambda i: (0,) * len(shape))


_deepfm_tc = pl.pallas_call(
    _deepfm_tc_body,
    grid=(B // BB,),
    in_specs=[
        pl.BlockSpec((BB, EMB_FLAT), lambda i: (i, 0)),
        pl.BlockSpec((BB, ND), lambda i: (i, 0)),
        _full((EMB_FLAT, 32)),
        _full((ND, 32)),
        _full((1, 32)),
        _full((32, 32)),
        _full((1, 32)),
        _full((EMB_FLAT, 1)),
        _full((ND, 1)),
        _full((1, 1)),
        _full((32, 1)),
        _full((1, 1)),
        _full((2, 1)),
        _full((1, 1)),
    ],
    out_specs=pl.BlockSpec((BB, 1), lambda i: (i, 0)),
    out_shape=jax.ShapeDtypeStruct((B, 1), jnp.float32),
)


def kernel(sparse_idx, dense_x, emb_tables, W1, b1, W2, b2, Wf, bf, Wd, bd, Wo, bo):
    v = sparse_idx.astype(jnp.int32)
    idx = (jnp.arange(NS, dtype=jnp.int32) * VOCAB)[None, :] + (
        8 * (v % ROWS_PER_F) + v // ROWS_PER_F)
    idx = idx.reshape(NW, NSUB, SUB)
    t3 = jnp.transpose(emb_tables, (0, 2, 1))
    table = _retile_tc(t3).reshape(NS * VOCAB, EMB)
    embs = _make_gather_sc()(table, idx)
    embs_flat = embs.reshape(B, EMB_FLAT)
    return _deepfm_tc(
        embs_flat, dense_x.astype(jnp.float32),
        W1[:EMB_FLAT], W1[EMB_FLAT:], b1.reshape(1, 32),
        W2, b2.reshape(1, 32),
        Wf[:EMB_FLAT], Wf[EMB_FLAT:], bf.reshape(1, 1),
        Wd, bd.reshape(1, 1), Wo, bo.reshape(1, 1))


# speedup vs baseline: 4.1580x; 4.1580x over previous
import functools

import jax
import jax.numpy as jnp
from jax import lax
from jax.experimental import pallas as pl
from jax.experimental.pallas import tpu as pltpu
from jax.experimental.pallas import tpu_sc as plsc

B = 16384
NS = 26
ND = 13
VOCAB = 100000
EMB = 16
EMB_FLAT = NS * EMB
NROWS = B * NS
NW = 32
ROWS_PER_W = NROWS // NW
SUB = 128
NSUB = ROWS_PER_W // SUB
GROUP = 13
CHUNK = SUB * GROUP
NCHUNK = ROWS_PER_W // CHUNK

@functools.cache
def _make_gather_sc():
    mesh = plsc.VectorSubcoreMesh(core_axis_name="c", subcore_axis_name="s")

    @functools.partial(
        pl.kernel,
        mesh=mesh,
        out_type=jax.ShapeDtypeStruct((NROWS, EMB), jnp.float32),
        compiler_params=pltpu.CompilerParams(use_tc_tiling_on_sc=False),
        scratch_types=[
            pltpu.VMEM((NSUB, SUB), jnp.int32),
            pltpu.VMEM((CHUNK, EMB), jnp.float32),
            pltpu.VMEM((CHUNK, EMB), jnp.float32),
            pltpu.SemaphoreType.DMA,
            pltpu.SemaphoreType.DMA,
            pltpu.SemaphoreType.DMA,
            pltpu.SemaphoreType.DMA,
        ],
    )
    def _gather_sc(table_hbm, idx_hbm, out_hbm, idx_v, buf0, buf1, g0, g1, o0, o1):
        wid = lax.axis_index("s") * 2 + lax.axis_index("c")
        base = wid * ROWS_PER_W
        pltpu.sync_copy(idx_hbm.at[wid], idx_v)
        bufs = (buf0, buf1)
        gsem = (g0, g1)
        osem = (o0, o1)
        def fire(grp, b):
            return [
                pltpu.async_copy(
                    table_hbm.at[idx_v.at[grp * GROUP + i]],
                    bufs[b].at[pl.ds(i * SUB, SUB)], gsem[b])
                for i in range(GROUP)
            ]

        gath = [None, None]
        outc = [None, None]
        gath[0] = fire(0, 0)
        for j in range(NCHUNK):
            cur = j & 1
            nxt = 1 - cur
            if j + 1 < NCHUNK:
                if outc[nxt] is not None:
                    outc[nxt].wait()
                gath[nxt] = fire(j + 1, nxt)
            for c in gath[cur]:
                c.wait()
            outc[cur] = pltpu.async_copy(
                bufs[cur], out_hbm.at[pl.ds(base + j * CHUNK, CHUNK)], osem[cur])
        outc[0].wait()
        outc[1].wait()

    return _gather_sc


VSUB = 10000
FLAT_ROWS = NS * VOCAB * EMB // 128
ROWS_PER_F = VOCAB * EMB // 128


def _retile_tc_body(x_ref, o_ref):
    r_i = jax.lax.broadcasted_iota(jnp.int32, (128, 128), 0)
    c_i = jax.lax.broadcasted_iota(jnp.int32, (128, 128), 1)
    eye = (r_i == c_i).astype(jnp.float32)
    for f01 in range(2):
        xr = jnp.concatenate(
            [x_ref[f01, :, m * ROWS_PER_F:(m + 1) * ROWS_PER_F]
             for m in range(8)], axis=0)
        o_ref[f01 * ROWS_PER_F:(f01 + 1) * ROWS_PER_F, :] = (
            jax.lax.dot_general(
                xr, eye, (((0,), (0,)), ((), ())),
                preferred_element_type=jnp.float32))


_retile_tc = pl.pallas_call(
    _retile_tc_body,
    grid=(NS // 2,),
    in_specs=[pl.BlockSpec((2, EMB, VOCAB), lambda k: (k, 0, 0))],
    out_specs=pl.BlockSpec((2 * ROWS_PER_F, 128), lambda k: (k, 0)),
    out_shape=jax.ShapeDtypeStruct((FLAT_ROWS, 128), jnp.float32),
    compiler_params=pltpu.CompilerParams(vmem_limit_bytes=100 * 1024 * 1024),
)


BB = 2048


def _deepfm_tc_body(embs_ref, dx_ref, w1e_ref, w1d_ref, b1_ref, w2_ref, b2_ref,
                    wfe_ref, wfd_ref, bf_ref, wd_ref, bd_ref, wo_ref, bo_ref,
                    out_ref):
    e = embs_ref[...]
    dx = dx_ref[...]
    x = jnp.dot(e, w1e_ref[...], preferred_element_type=jnp.float32)
    x = x + jnp.dot(dx, w1d_ref[...], preferred_element_type=jnp.float32)
    x = jnp.maximum(x + b1_ref[...], 0.0)
    x = jnp.dot(x, w2_ref[...], preferred_element_type=jnp.float32)
    x = jnp.maximum(x + b2_ref[...], 0.0)
    fm = jnp.dot(e, wfe_ref[...], preferred_element_type=jnp.float32)
    fm = fm + jnp.dot(dx, wfd_ref[...], preferred_element_type=jnp.float32)
    fm = fm + bf_ref[...]
    deep = jnp.dot(x, wd_ref[...], preferred_element_type=jnp.float32)
    deep = deep + bd_ref[...]
    wo = wo_ref[...]
    z = fm * wo[0:1, :] + deep * wo[1:2, :] + bo_ref[...]
    out_ref[...] = jax.nn.sigmoid(z)


def _full(shape):
    return pl.BlockSpec(shape, lambda i: (0,) * len(shape))


_deepfm_tc = pl.pallas_call(
    _deepfm_tc_body,
    grid=(B // BB,),
    in_specs=[
        pl.BlockSpec((BB, EMB_FLAT), lambda i: (i, 0)),
        pl.BlockSpec((BB, ND), lambda i: (i, 0)),
        _full((EMB_FLAT, 32)),
        _full((ND, 32)),
        _full((1, 32)),
        _full((32, 32)),
        _full((1, 32)),
        _full((EMB_FLAT, 1)),
        _full((ND, 1)),
        _full((1, 1)),
        _full((32, 1)),
        _full((1, 1)),
        _full((2, 1)),
        _full((1, 1)),
    ],
    out_specs=pl.BlockSpec((BB, 1), lambda i: (i, 0)),
    out_shape=jax.ShapeDtypeStruct((B, 1), jnp.float32),
)


def kernel(sparse_idx, dense_x, emb_tables, W1, b1, W2, b2, Wf, bf, Wd, bd, Wo, bo):
    v = sparse_idx.astype(jnp.int32)
    idx = (jnp.arange(NS, dtype=jnp.int32) * VOCAB)[None, :] + (
        8 * (v % ROWS_PER_F) + v // ROWS_PER_F)
    idx = idx.reshape(NW, NSUB, SUB)
    t3 = jnp.transpose(emb_tables, (0, 2, 1))
    table = _retile_tc(t3).reshape(NS * VOCAB, EMB)
    embs = _make_gather_sc()(table, idx)
    embs_flat = embs.reshape(B, EMB_FLAT)
    return _deepfm_tc(
        embs_flat, dense_x.astype(jnp.float32),
        W1[:EMB_FLAT], W1[EMB_FLAT:], b1.reshape(1, 32),
        W2, b2.reshape(1, 32),
        Wf[:EMB_FLAT], Wf[EMB_FLAT:], bf.reshape(1, 1),
        Wd, bd.reshape(1, 1), Wo, bo.reshape(1, 1))


# speedup vs baseline: 4.1605x; 1.0006x over previous
import functools

import jax
import jax.numpy as jnp
from jax import lax
from jax.experimental import pallas as pl
from jax.experimental.pallas import tpu as pltpu
from jax.experimental.pallas import tpu_sc as plsc

B = 16384
NS = 26
ND = 13
VOCAB = 100000
EMB = 16
EMB_FLAT = NS * EMB
NROWS = B * NS
NW = 32
ROWS_PER_W = NROWS // NW
SUB = 128
NSUB = ROWS_PER_W // SUB
GROUP = 13
CHUNK = SUB * GROUP
NCHUNK = ROWS_PER_W // CHUNK

@functools.cache
def _make_gather_sc():
    mesh = plsc.VectorSubcoreMesh(core_axis_name="c", subcore_axis_name="s")

    @functools.partial(
        pl.kernel,
        mesh=mesh,
        out_type=jax.ShapeDtypeStruct((NROWS, EMB), jnp.float32),
        compiler_params=pltpu.CompilerParams(use_tc_tiling_on_sc=False),
        scratch_types=[
            pltpu.VMEM((NSUB, SUB), jnp.int32),
            pltpu.VMEM((CHUNK, EMB), jnp.float32),
            pltpu.VMEM((CHUNK, EMB), jnp.float32),
            pltpu.SemaphoreType.DMA,
            pltpu.SemaphoreType.DMA,
            pltpu.SemaphoreType.DMA,
            pltpu.SemaphoreType.DMA,
        ],
    )
    def _gather_sc(table_hbm, idx_hbm, out_hbm, idx_v, buf0, buf1, g0, g1, o0, o1):
        wid = lax.axis_index("s") * 2 + lax.axis_index("c")
        base = wid * ROWS_PER_W
        pltpu.sync_copy(idx_hbm.at[wid], idx_v)
        bufs = (buf0, buf1)
        gsem = (g0, g1)
        osem = (o0, o1)
        def fire(grp, b):
            return [
                pltpu.async_copy(
                    table_hbm.at[idx_v.at[grp * GROUP + i]],
                    bufs[b].at[pl.ds(i * SUB, SUB)], gsem[b])
                for i in range(GROUP)
            ]

        gath = [None, None]
        outc = [None, None]
        gath[0] = fire(0, 0)
        for j in range(NCHUNK):
            cur = j & 1
            nxt = 1 - cur
            if j + 1 < NCHUNK:
                if outc[nxt] is not None:
                    outc[nxt].wait()
                gath[nxt] = fire(j + 1, nxt)
            for c in gath[cur]:
                c.wait()
            outc[cur] = pltpu.async_copy(
                bufs[cur], out_hbm.at[pl.ds(base + j * CHUNK, CHUNK)], osem[cur])
        outc[0].wait()
        outc[1].wait()

    return _gather_sc


FLAT_ROWS = NS * VOCAB * EMB // 128
ROWS_PER_F = VOCAB * EMB // 128


def _retile_tc_body(x_ref, o_ref):
    r_i = jax.lax.broadcasted_iota(jnp.int32, (128, 128), 0)
    c_i = jax.lax.broadcasted_iota(jnp.int32, (128, 128), 1)
    eye = (r_i == c_i).astype(jnp.float32)
    for f01 in range(2):
        xr = jnp.concatenate(
            [x_ref[f01, :, m * ROWS_PER_F:(m + 1) * ROWS_PER_F]
             for m in range(8)], axis=0)
        o_ref[f01 * ROWS_PER_F:(f01 + 1) * ROWS_PER_F, :] = (
            jax.lax.dot_general(
                xr, eye, (((0,), (0,)), ((), ())),
                preferred_element_type=jnp.float32))


_retile_tc = pl.pallas_call(
    _retile_tc_body,
    grid=(NS // 2,),
    in_specs=[pl.BlockSpec((2, EMB, VOCAB), lambda k: (k, 0, 0))],
    out_specs=pl.BlockSpec((2 * ROWS_PER_F, 128), lambda k: (k, 0)),
    out_shape=jax.ShapeDtypeStruct((FLAT_ROWS, 128), jnp.float32),
    compiler_params=pltpu.CompilerParams(vmem_limit_bytes=100 * 1024 * 1024),
)


BB = 2048


def _deepfm_tc_body(embs_ref, dx_ref, w1e_ref, w1d_ref, b1_ref, w2_ref, b2_ref,
                    wfe_ref, wfd_ref, bf_ref, wd_ref, bd_ref, wo_ref, bo_ref,
                    out_ref):
    e = embs_ref[...]
    dx = dx_ref[...]
    x = jnp.dot(e, w1e_ref[...], preferred_element_type=jnp.float32)
    x = x + jnp.dot(dx, w1d_ref[...], preferred_element_type=jnp.float32)
    x = jnp.maximum(x + b1_ref[...], 0.0)
    x = jnp.dot(x, w2_ref[...], preferred_element_type=jnp.float32)
    x = jnp.maximum(x + b2_ref[...], 0.0)
    fm = jnp.dot(e, wfe_ref[...], preferred_element_type=jnp.float32)
    fm = fm + jnp.dot(dx, wfd_ref[...], preferred_element_type=jnp.float32)
    fm = fm + bf_ref[...]
    deep = jnp.dot(x, wd_ref[...], preferred_element_type=jnp.float32)
    deep = deep + bd_ref[...]
    wo = wo_ref[...]
    z = fm * wo[0:1, :] + deep * wo[1:2, :] + bo_ref[...]
    out_ref[...] = jax.nn.sigmoid(z)


def _full(shape):
    return pl.BlockSpec(shape, lambda i: (0,) * len(shape))


_deepfm_tc = pl.pallas_call(
    _deepfm_tc_body,
    grid=(B // BB,),
    in_specs=[
        pl.BlockSpec((BB, EMB_FLAT), lambda i: (i, 0)),
        pl.BlockSpec((BB, ND), lambda i: (i, 0)),
        _full((EMB_FLAT, 32)),
        _full((ND, 32)),
        _full((1, 32)),
        _full((32, 32)),
        _full((1, 32)),
        _full((EMB_FLAT, 1)),
        _full((ND, 1)),
        _full((1, 1)),
        _full((32, 1)),
        _full((1, 1)),
        _full((2, 1)),
        _full((1, 1)),
    ],
    out_specs=pl.BlockSpec((BB, 1), lambda i: (i, 0)),
    out_shape=jax.ShapeDtypeStruct((B, 1), jnp.float32),
)


def kernel(sparse_idx, dense_x, emb_tables, W1, b1, W2, b2, Wf, bf, Wd, bd, Wo, bo):
    v = sparse_idx.astype(jnp.int32)
    idx = (jnp.arange(NS, dtype=jnp.int32) * VOCAB)[None, :] + (
        8 * (v % ROWS_PER_F) + v // ROWS_PER_F)
    idx = idx.reshape(NW, NSUB, SUB)
    t3 = jnp.transpose(emb_tables, (0, 2, 1))
    table = _retile_tc(t3).reshape(NS * VOCAB, EMB)
    embs = _make_gather_sc()(table, idx)
    embs_flat = embs.reshape(B, EMB_FLAT)
    return _deepfm_tc(
        embs_flat, dense_x.astype(jnp.float32),
        W1[:EMB_FLAT], W1[EMB_FLAT:], b1.reshape(1, 32),
        W2, b2.reshape(1, 32),
        Wf[:EMB_FLAT], Wf[EMB_FLAT:], bf.reshape(1, 1),
        Wd, bd.reshape(1, 1), Wo, bo.reshape(1, 1))
